# baseline (speedup 1.0000x reference)
import functools

import jax
import jax.numpy as jnp
from jax import lax
from jax.experimental import pallas as pl
from jax.experimental.pallas import tpu as pltpu
from jax.experimental.pallas import tpu_sc as plsc

_L = 16


def _pack_table(emb_t, V, D):
    C = 512
    nblk = (V + 2 * C - 1) // (2 * C)
    H = nblk * C

    def body(a_ref, b_ref, out_ref):
        eye = jnp.eye(D, dtype=jnp.float32)
        dn = (((0,), (0,)), ((), ()))
        at = lax.dot_general(a_ref[...], eye, dn,
                             precision=lax.Precision.HIGHEST,
                             preferred_element_type=jnp.float32)
        bt = lax.dot_general(b_ref[...], eye, dn,
                             precision=lax.Precision.HIGHEST,
                             preferred_element_type=jnp.float32)
        out_ref[...] = jnp.concatenate([at, bt], axis=1)

    packed = pl.pallas_call(
        body,
        grid=(nblk,),
        in_specs=[pl.BlockSpec((D, C), lambda c: (0, c)),
                  pl.BlockSpec((D, C), lambda c: (0, c + nblk))],
        out_specs=pl.BlockSpec((C, 2 * D), lambda c: (c, 0)),
        out_shape=jax.ShapeDtypeStruct((H, 2 * D), jnp.float32),
        compiler_params=pltpu.CompilerParams(
            fuse_transposed_lhs_in_matmul=True),
    )(emb_t, emb_t)
    return packed, H


def _sc_dots(center, target, negr, v_pack, u_pack, B, K, D, H, nw, nc):
    bpw = B // nw
    S = 8
    nsub = bpw // S
    spk = S * K
    gpk = spk // _L
    gsizes = []
    rem = spk
    while rem > 0:
        gsizes.append(min(128, rem))
        rem -= gsizes[-1]
    nchunk = D // _L
    Dp = 2 * D

    mesh = plsc.VectorSubcoreMesh(core_axis_name="c", subcore_axis_name="s")

    @functools.partial(
        pl.kernel,
        mesh=mesh,
        compiler_params=pltpu.CompilerParams(needs_layout_passes=False),
        out_type=[
            jax.ShapeDtypeStruct((B,), jnp.float32),
            jax.ShapeDtypeStruct((B * K,), jnp.float32),
        ],
        scratch_types=[
            pltpu.VMEM((bpw,), jnp.int32),
            pltpu.VMEM((bpw + _L,), jnp.int32),
            pltpu.VMEM((bpw,), jnp.int32),
            pltpu.VMEM((bpw + _L,), jnp.int32),
            pltpu.VMEM((bpw * K,), jnp.int32),
            pltpu.VMEM((bpw * K + _L,), jnp.int32),
            pltpu.VMEM((bpw, Dp), jnp.float32),
            pltpu.VMEM((bpw, Dp), jnp.float32),
            pltpu.VMEM((spk, Dp), jnp.float32),
            pltpu.VMEM((spk, Dp), jnp.float32),
            pltpu.VMEM((spk, _L), jnp.float32),
            pltpu.VMEM((bpw,), jnp.float32),
            pltpu.VMEM((spk,), jnp.float32),
            pltpu.VMEM((spk,), jnp.float32),
            pltpu.SemaphoreType.DMA,
            pltpu.SemaphoreType.DMA,
            pltpu.SemaphoreType.DMA,
            pltpu.SemaphoreType.DMA,
            pltpu.SemaphoreType.DMA,
            pltpu.SemaphoreType.DMA,
        ],
    )
    def sc_kernel(center_hbm, target_hbm, negr_hbm, vemb_hbm, uemb_hbm,
                  pos_hbm, negout_hbm,
                  cidx, cpar, tidx, tpar, nidx, npar,
                  vrows, urows, nrows0, nrows1,
                  accb, posd, negd0, negd1,
                  semi, semv, semu, semn0, semn1, semd):
        nrows = (nrows0, nrows1)
        negd = (negd0, negd1)
        nsem = (semn0, semn1)
        wid = lax.axis_index("s") * nc + lax.axis_index("c")
        base = wid * bpw
        lane = lax.iota(jnp.int32, _L)

        def split_idx(idx_ref, par_ref, n):
            def it(i, carry):
                v = idx_ref[pl.ds(i * _L, _L)]
                hi = v >= H
                par_ref[pl.ds(i * _L, _L)] = jnp.where(hi, D, 0)
                idx_ref[pl.ds(i * _L, _L)] = jnp.where(hi, v - H, v)
                return carry
            lax.fori_loop(0, n // _L, it, 0)

        hc = pltpu.async_copy(center_hbm.at[pl.ds(base, bpw)], cidx, semi)
        ht = pltpu.async_copy(target_hbm.at[pl.ds(base, bpw)], tidx, semi)
        hn_idx = pltpu.async_copy(
            negr_hbm.at[pl.ds(base * K, bpw * K)], nidx, semi)
        hc.wait()
        ht.wait()
        split_idx(cidx, cpar, bpw)
        split_idx(tidx, tpar, bpw)
        hv = pltpu.async_copy(vemb_hbm.at[cidx], vrows, semv)
        hu = pltpu.async_copy(uemb_hbm.at[tidx], urows, semu)
        hn_idx.wait()
        split_idx(nidx, npar, bpw * K)

        def fire_neg_gathers(t, slot):
            off = 0
            for g in gsizes:
                pltpu.async_copy(
                    uemb_hbm.at[nidx.at[pl.ds(t * spk + off, g)]],
                    nrows[slot].at[pl.ds(off, g)],
                    nsem[slot])
                off += g

        def drain_neg_gathers(slot):
            off = 0
            for g in gsizes:
                pltpu.make_async_copy(
                    uemb_hbm.at[pl.ds(0, g)],
                    nrows[slot].at[pl.ds(off, g)],
                    nsem[slot]).wait()
                off += g

        fire_neg_gathers(0, 0)

        def reduce_groups(ngroups, out_ref):
            def group(g, carry):
                rowi = lane + g * _L
                tot = plsc.load_gather(accb, [rowi, jnp.zeros((_L,), jnp.int32)])
                for l in range(1, _L):
                    tot = tot + plsc.load_gather(
                        accb, [rowi, jnp.full((_L,), l, jnp.int32)])
                out_ref[pl.ds(g * _L, _L)] = tot
                return carry
            lax.fori_loop(0, ngroups, group, 0)

        hv.wait()
        hu.wait()

        def pos_row(r, carry):
            cb = cpar[pl.ds(r, _L)][0]
            tb = tpar[pl.ds(r, _L)][0]
            acc = (urows[r, pl.ds(tb, _L)] * vrows[r, pl.ds(cb, _L)])
            for c in range(1, nchunk):
                acc = acc + (urows[r, pl.ds(tb + c * _L, _L)]
                             * vrows[r, pl.ds(cb + c * _L, _L)])
            accb[r, :] = acc
            return carry
        lax.fori_loop(0, bpw, pos_row, 0)
        reduce_groups(bpw // _L, posd)
        pltpu.sync_copy(posd, pos_hbm.at[pl.ds(base, bpw)])

        def half(t, slot):
            @pl.when(t + 1 < nsub)
            def _():
                fire_neg_gathers(t + 1, 1 - slot)
            drain_neg_gathers(slot)

            @pl.when(t >= 2)
            def _():
                pltpu.make_async_copy(
                    negd[slot], negout_hbm.at[pl.ds(base * K, spk)],
                    semd).wait()

            def neg_row(j, carry):
                rr = t * S + j
                cb = cpar[pl.ds(rr, _L)][0]
                vc = [vrows[rr, pl.ds(cb + c * _L, _L)] for c in range(nchunk)]
                parv = [npar[pl.ds((t * S + j) * K + 16 * u, _L)]
                        for u in range((K + _L - 1) // _L)]
                for k in range(K):
                    row = j * K + k
                    nb = parv[k // _L][k % _L]
                    acc = nrows[slot][row, pl.ds(nb, _L)] * vc[0]
                    for c in range(1, nchunk):
                        acc = acc + (nrows[slot][row, pl.ds(nb + c * _L, _L)]
                                     * vc[c])
                    accb[row, :] = acc
                return carry
            lax.fori_loop(0, S, neg_row, 0)

            reduce_groups(gpk, negd[slot])
            pltpu.async_copy(
                negd[slot],
                negout_hbm.at[pl.ds((base + t * S) * K, spk)],
                semd)

        def pair(s2, carry):
            half(2 * s2, 0)
            half(2 * s2 + 1, 1)
            return carry
        lax.fori_loop(0, nsub // 2, pair, 0)

        for _ in range(2):
            pltpu.make_async_copy(
                negd[0], negout_hbm.at[pl.ds(base * K, spk)], semd).wait()

    return sc_kernel(center, target, negr, v_pack, u_pack)


def _tc_loss(pos2d, neg2d, B):
    def body(pos_ref, neg_ref, o_ref):
        lp = jax.nn.log_sigmoid(pos_ref[...])
        ln = jax.nn.log_sigmoid(-neg_ref[...])
        o_ref[0, 0] = -(jnp.sum(lp) + jnp.sum(ln)) / jnp.float32(B)

    out = pl.pallas_call(
        body,
        out_shape=jax.ShapeDtypeStruct((1, 1), jnp.float32),
        out_specs=pl.BlockSpec(memory_space=pltpu.SMEM),
    )(pos2d, neg2d)
    return out[0, 0]


def kernel(center, target, negative, v_emb, u_emb):
    B = center.shape[0]
    V, D = v_emb.shape
    K = negative.shape[1]

    info = plsc.get_sparse_core_info()
    nc, ns = info.num_cores, info.num_subcores
    nw = nc * ns

    center = center.astype(jnp.int32)
    target = target.astype(jnp.int32)
    negr = negative.astype(jnp.int32).reshape(B * K)

    v_pack, H = _pack_table(v_emb.T, V, D)
    u_pack, _ = _pack_table(u_emb.T, V, D)

    pos, negdots = _sc_dots(center, target, negr, v_pack, u_pack,
                            B, K, D, H, nw, nc)

    pos2d = pos.reshape(B // 128, 128)
    neg2d = negdots.reshape(B * K // 128, 128)
    return _tc_loss(pos2d, neg2d, B)


# speedup vs baseline: 1.2174x; 1.2174x over previous
import functools

import jax
import jax.numpy as jnp
from jax import lax
from jax.experimental import pallas as pl
from jax.experimental.pallas import tpu as pltpu
from jax.experimental.pallas import tpu_sc as plsc

_L = 16


def _pack_table(emb_t, V, D):
    C = 512
    nblk = (V + 2 * C - 1) // (2 * C)
    H = nblk * C

    def body(a_ref, b_ref, out_ref):
        eye = jnp.eye(D, dtype=jnp.float32)
        dn = (((0,), (0,)), ((), ()))
        at = lax.dot_general(a_ref[...], eye, dn,
                             preferred_element_type=jnp.float32)
        bt = lax.dot_general(b_ref[...], eye, dn,
                             preferred_element_type=jnp.float32)
        out_ref[...] = jnp.concatenate([at, bt], axis=1)

    packed = pl.pallas_call(
        body,
        grid=(nblk,),
        in_specs=[pl.BlockSpec((D, C), lambda c: (0, c)),
                  pl.BlockSpec((D, C), lambda c: (0, c + nblk))],
        out_specs=pl.BlockSpec((C, 2 * D), lambda c: (c, 0)),
        out_shape=jax.ShapeDtypeStruct((H, 2 * D), jnp.float32),
        compiler_params=pltpu.CompilerParams(
            fuse_transposed_lhs_in_matmul=True),
    )(emb_t, emb_t)
    return packed, H


def _sc_dots(center, target, negr, v_pack, u_pack, B, K, D, H, nw, nc):
    bpw = B // nw
    S = 8
    nsub = bpw // S
    spk = S * K
    gpk = spk // _L
    gsizes = []
    rem = spk
    while rem > 0:
        gsizes.append(min(128, rem))
        rem -= gsizes[-1]
    nchunk = D // _L
    Dp = 2 * D

    mesh = plsc.VectorSubcoreMesh(core_axis_name="c", subcore_axis_name="s")

    @functools.partial(
        pl.kernel,
        mesh=mesh,
        compiler_params=pltpu.CompilerParams(needs_layout_passes=False),
        out_type=[
            jax.ShapeDtypeStruct((B,), jnp.float32),
            jax.ShapeDtypeStruct((B * K,), jnp.float32),
        ],
        scratch_types=[
            pltpu.VMEM((bpw,), jnp.int32),
            pltpu.VMEM((bpw + _L,), jnp.int32),
            pltpu.VMEM((bpw,), jnp.int32),
            pltpu.VMEM((bpw + _L,), jnp.int32),
            pltpu.VMEM((bpw * K,), jnp.int32),
            pltpu.VMEM((bpw * K + _L,), jnp.int32),
            pltpu.VMEM((bpw, Dp), jnp.float32),
            pltpu.VMEM((bpw, Dp), jnp.float32),
            pltpu.VMEM((spk, Dp), jnp.float32),
            pltpu.VMEM((spk, Dp), jnp.float32),
            pltpu.VMEM((spk, _L), jnp.float32),
            pltpu.VMEM((bpw,), jnp.float32),
            pltpu.VMEM((spk,), jnp.float32),
            pltpu.VMEM((spk,), jnp.float32),
            pltpu.SemaphoreType.DMA,
            pltpu.SemaphoreType.DMA,
            pltpu.SemaphoreType.DMA,
            pltpu.SemaphoreType.DMA,
            pltpu.SemaphoreType.DMA,
            pltpu.SemaphoreType.DMA,
        ],
    )
    def sc_kernel(center_hbm, target_hbm, negr_hbm, vemb_hbm, uemb_hbm,
                  pos_hbm, negout_hbm,
                  cidx, cpar, tidx, tpar, nidx, npar,
                  vrows, urows, nrows0, nrows1,
                  accb, posd, negd0, negd1,
                  semi, semv, semu, semn0, semn1, semd):
        nrows = (nrows0, nrows1)
        negd = (negd0, negd1)
        nsem = (semn0, semn1)
        wid = lax.axis_index("s") * nc + lax.axis_index("c")
        base = wid * bpw
        lane = lax.iota(jnp.int32, _L)

        def split_idx(idx_ref, par_ref, n):
            def it(i, carry):
                v = idx_ref[pl.ds(i * _L, _L)]
                hi = v >= H
                par_ref[pl.ds(i * _L, _L)] = jnp.where(hi, D, 0)
                idx_ref[pl.ds(i * _L, _L)] = jnp.where(hi, v - H, v)
                return carry
            lax.fori_loop(0, n // _L, it, 0)

        hc = pltpu.async_copy(center_hbm.at[pl.ds(base, bpw)], cidx, semi)
        ht = pltpu.async_copy(target_hbm.at[pl.ds(base, bpw)], tidx, semi)
        hn_idx = pltpu.async_copy(
            negr_hbm.at[pl.ds(base * K, bpw * K)], nidx, semi)
        hc.wait()
        ht.wait()
        split_idx(cidx, cpar, bpw)
        split_idx(tidx, tpar, bpw)
        hv = pltpu.async_copy(vemb_hbm.at[cidx], vrows, semv)
        hu = pltpu.async_copy(uemb_hbm.at[tidx], urows, semu)
        hn_idx.wait()
        split_idx(nidx, npar, bpw * K)

        def fire_neg_gathers(t, slot):
            off = 0
            for g in gsizes:
                pltpu.async_copy(
                    uemb_hbm.at[nidx.at[pl.ds(t * spk + off, g)]],
                    nrows[slot].at[pl.ds(off, g)],
                    nsem[slot])
                off += g

        def drain_neg_gathers(slot):
            off = 0
            for g in gsizes:
                pltpu.make_async_copy(
                    uemb_hbm.at[pl.ds(0, g)],
                    nrows[slot].at[pl.ds(off, g)],
                    nsem[slot]).wait()
                off += g

        fire_neg_gathers(0, 0)

        def reduce_groups(ngroups, out_ref):
            def group(g, carry):
                rowi = lane + g * _L
                tot = plsc.load_gather(accb, [rowi, jnp.zeros((_L,), jnp.int32)])
                for l in range(1, _L):
                    tot = tot + plsc.load_gather(
                        accb, [rowi, jnp.full((_L,), l, jnp.int32)])
                out_ref[pl.ds(g * _L, _L)] = tot
                return carry
            lax.fori_loop(0, ngroups, group, 0)

        hv.wait()
        hu.wait()

        def pos_row(r, carry):
            cb = cpar[pl.ds(r, _L)][0]
            tb = tpar[pl.ds(r, _L)][0]
            acc = (urows[r, pl.ds(tb, _L)] * vrows[r, pl.ds(cb, _L)])
            for c in range(1, nchunk):
                acc = acc + (urows[r, pl.ds(tb + c * _L, _L)]
                             * vrows[r, pl.ds(cb + c * _L, _L)])
            accb[r, :] = acc
            return carry
        lax.fori_loop(0, bpw, pos_row, 0)
        reduce_groups(bpw // _L, posd)
        pltpu.sync_copy(posd, pos_hbm.at[pl.ds(base, bpw)])

        def half(t, slot):
            @pl.when(t + 1 < nsub)
            def _():
                fire_neg_gathers(t + 1, 1 - slot)
            drain_neg_gathers(slot)

            @pl.when(t >= 2)
            def _():
                pltpu.make_async_copy(
                    negd[slot], negout_hbm.at[pl.ds(base * K, spk)],
                    semd).wait()

            def neg_row(j, carry):
                rr = t * S + j
                cb = cpar[pl.ds(rr, _L)][0]
                vc = [vrows[rr, pl.ds(cb + c * _L, _L)] for c in range(nchunk)]
                parv = [npar[pl.ds((t * S + j) * K + 16 * u, _L)]
                        for u in range((K + _L - 1) // _L)]
                for k in range(K):
                    row = j * K + k
                    nb = parv[k // _L][k % _L]
                    acc = nrows[slot][row, pl.ds(nb, _L)] * vc[0]
                    for c in range(1, nchunk):
                        acc = acc + (nrows[slot][row, pl.ds(nb + c * _L, _L)]
                                     * vc[c])
                    accb[row, :] = acc
                return carry
            lax.fori_loop(0, S, neg_row, 0)

            reduce_groups(gpk, negd[slot])
            pltpu.async_copy(
                negd[slot],
                negout_hbm.at[pl.ds((base + t * S) * K, spk)],
                semd)

        def pair(s2, carry):
            half(2 * s2, 0)
            half(2 * s2 + 1, 1)
            return carry
        lax.fori_loop(0, nsub // 2, pair, 0)

        for _ in range(2):
            pltpu.make_async_copy(
                negd[0], negout_hbm.at[pl.ds(base * K, spk)], semd).wait()

    return sc_kernel(center, target, negr, v_pack, u_pack)


def _tc_loss(pos2d, neg2d, B):
    def body(pos_ref, neg_ref, o_ref):
        lp = jax.nn.log_sigmoid(pos_ref[...])
        ln = jax.nn.log_sigmoid(-neg_ref[...])
        o_ref[0, 0] = -(jnp.sum(lp) + jnp.sum(ln)) / jnp.float32(B)

    out = pl.pallas_call(
        body,
        out_shape=jax.ShapeDtypeStruct((1, 1), jnp.float32),
        out_specs=pl.BlockSpec(memory_space=pltpu.SMEM),
    )(pos2d, neg2d)
    return out[0, 0]


def kernel(center, target, negative, v_emb, u_emb):
    B = center.shape[0]
    V, D = v_emb.shape
    K = negative.shape[1]

    info = plsc.get_sparse_core_info()
    nc, ns = info.num_cores, info.num_subcores
    nw = nc * ns

    center = center.astype(jnp.int32)
    target = target.astype(jnp.int32)
    negr = negative.astype(jnp.int32).reshape(B * K)

    v_pack, H = _pack_table(v_emb.T, V, D)
    u_pack, _ = _pack_table(u_emb.T, V, D)

    pos, negdots = _sc_dots(center, target, negr, v_pack, u_pack,
                            B, K, D, H, nw, nc)

    pos2d = pos.reshape(B // 128, 128)
    neg2d = negdots.reshape(B * K // 128, 128)
    return _tc_loss(pos2d, neg2d, B)


# speedup vs baseline: 1.7042x; 1.3999x over previous
import functools

import jax
import jax.numpy as jnp
from jax import lax
from jax.experimental import pallas as pl
from jax.experimental.pallas import tpu as pltpu
from jax.experimental.pallas import tpu_sc as plsc

_L = 16


def _sc_dots(center, target, negr, v_pack, u_pack, B, K, D, H, nw, nc):
    bpw = B // nw
    S = 8
    nsub = bpw // S
    spk = S * K
    gpk = spk // _L
    gsizes = []
    rem = spk
    while rem > 0:
        gsizes.append(min(128, rem))
        rem -= gsizes[-1]
    nchunk = D // _L
    Dp = 2 * D

    mesh = plsc.VectorSubcoreMesh(core_axis_name="c", subcore_axis_name="s")

    @functools.partial(
        pl.kernel,
        mesh=mesh,
        compiler_params=pltpu.CompilerParams(needs_layout_passes=False),
        out_type=[
            jax.ShapeDtypeStruct((B,), jnp.float32),
            jax.ShapeDtypeStruct((B * K,), jnp.float32),
        ],
        scratch_types=[
            pltpu.VMEM((bpw,), jnp.int32),
            pltpu.VMEM((bpw + _L,), jnp.int32),
            pltpu.VMEM((bpw,), jnp.int32),
            pltpu.VMEM((bpw + _L,), jnp.int32),
            pltpu.VMEM((bpw * K,), jnp.int32),
            pltpu.VMEM((bpw * K + _L,), jnp.int32),
            pltpu.VMEM((bpw, Dp), jnp.float32),
            pltpu.VMEM((bpw, Dp), jnp.float32),
            pltpu.VMEM((spk, Dp), jnp.float32),
            pltpu.VMEM((spk, Dp), jnp.float32),
            pltpu.VMEM((spk, _L), jnp.float32),
            pltpu.VMEM((bpw,), jnp.float32),
            pltpu.VMEM((spk,), jnp.float32),
            pltpu.VMEM((spk,), jnp.float32),
            pltpu.SemaphoreType.DMA,
            pltpu.SemaphoreType.DMA,
            pltpu.SemaphoreType.DMA,
            pltpu.SemaphoreType.DMA,
            pltpu.SemaphoreType.DMA,
            pltpu.SemaphoreType.DMA,
        ],
    )
    def sc_kernel(center_hbm, target_hbm, negr_hbm, vemb_hbm, uemb_hbm,
                  pos_hbm, negout_hbm,
                  cidx, cpar, tidx, tpar, nidx, npar,
                  vrows, urows, nrows0, nrows1,
                  accb, posd, negd0, negd1,
                  semi, semv, semu, semn0, semn1, semd):
        nrows = (nrows0, nrows1)
        negd = (negd0, negd1)
        nsem = (semn0, semn1)
        wid = lax.axis_index("s") * nc + lax.axis_index("c")
        base = wid * bpw
        lane = lax.iota(jnp.int32, _L)

        def split_idx(idx_ref, par_ref, n):
            def it(i, carry):
                v = idx_ref[pl.ds(i * _L, _L)]
                hi = v >= H
                par_ref[pl.ds(i * _L, _L)] = jnp.where(hi, D, 0)
                idx_ref[pl.ds(i * _L, _L)] = jnp.where(hi, v - H, v)
                return carry
            lax.fori_loop(0, n // _L, it, 0)

        hc = pltpu.async_copy(center_hbm.at[pl.ds(base, bpw)], cidx, semi)
        ht = pltpu.async_copy(target_hbm.at[pl.ds(base, bpw)], tidx, semi)
        hn_idx = pltpu.async_copy(
            negr_hbm.at[pl.ds(base * K, bpw * K)], nidx, semi)
        hc.wait()
        ht.wait()
        split_idx(cidx, cpar, bpw)
        split_idx(tidx, tpar, bpw)
        hv = pltpu.async_copy(vemb_hbm.at[cidx], vrows, semv)
        hu = pltpu.async_copy(uemb_hbm.at[tidx], urows, semu)
        hn_idx.wait()
        split_idx(nidx, npar, bpw * K)

        def fire_neg_gathers(t, slot):
            off = 0
            for g in gsizes:
                pltpu.async_copy(
                    uemb_hbm.at[nidx.at[pl.ds(t * spk + off, g)]],
                    nrows[slot].at[pl.ds(off, g)],
                    nsem[slot])
                off += g

        def drain_neg_gathers(slot):
            off = 0
            for g in gsizes:
                pltpu.make_async_copy(
                    uemb_hbm.at[pl.ds(0, g)],
                    nrows[slot].at[pl.ds(off, g)],
                    nsem[slot]).wait()
                off += g

        fire_neg_gathers(0, 0)

        def reduce_groups(ngroups, out_ref):
            def group(g, carry):
                rowi = lane + g * _L
                tot = plsc.load_gather(accb, [rowi, jnp.zeros((_L,), jnp.int32)])
                for l in range(1, _L):
                    tot = tot + plsc.load_gather(
                        accb, [rowi, jnp.full((_L,), l, jnp.int32)])
                out_ref[pl.ds(g * _L, _L)] = tot
                return carry
            lax.fori_loop(0, ngroups, group, 0)

        hv.wait()
        hu.wait()

        def pos_row(r, carry):
            cb = cpar[pl.ds(r, _L)][0]
            tb = tpar[pl.ds(r, _L)][0]
            acc = (urows[r, pl.ds(tb, _L)] * vrows[r, pl.ds(cb, _L)])
            for c in range(1, nchunk):
                acc = acc + (urows[r, pl.ds(tb + c * _L, _L)]
                             * vrows[r, pl.ds(cb + c * _L, _L)])
            accb[r, :] = acc
            return carry
        lax.fori_loop(0, bpw, pos_row, 0)
        reduce_groups(bpw // _L, posd)
        pltpu.sync_copy(posd, pos_hbm.at[pl.ds(base, bpw)])

        def half(t, slot):
            @pl.when(t + 1 < nsub)
            def _():
                fire_neg_gathers(t + 1, 1 - slot)
            drain_neg_gathers(slot)

            @pl.when(t >= 2)
            def _():
                pltpu.make_async_copy(
                    negd[slot], negout_hbm.at[pl.ds(base * K, spk)],
                    semd).wait()

            def neg_row(j, carry):
                rr = t * S + j
                cb = cpar[pl.ds(rr, _L)][0]
                vc = [vrows[rr, pl.ds(cb + c * _L, _L)] for c in range(nchunk)]
                parv = [npar[pl.ds((t * S + j) * K + 16 * u, _L)]
                        for u in range((K + _L - 1) // _L)]
                for k in range(K):
                    row = j * K + k
                    nb = parv[k // _L][k % _L]
                    acc = nrows[slot][row, pl.ds(nb, _L)] * vc[0]
                    for c in range(1, nchunk):
                        acc = acc + (nrows[slot][row, pl.ds(nb + c * _L, _L)]
                                     * vc[c])
                    accb[row, :] = acc
                return carry
            lax.fori_loop(0, S, neg_row, 0)

            reduce_groups(gpk, negd[slot])
            pltpu.async_copy(
                negd[slot],
                negout_hbm.at[pl.ds((base + t * S) * K, spk)],
                semd)

        def pair(s2, carry):
            half(2 * s2, 0)
            half(2 * s2 + 1, 1)
            return carry
        lax.fori_loop(0, nsub // 2, pair, 0)

        for _ in range(2):
            pltpu.make_async_copy(
                negd[0], negout_hbm.at[pl.ds(base * K, spk)], semd).wait()

    return sc_kernel(center, target, negr, v_pack, u_pack)


def _tc_loss(pos2d, neg2d, B):
    def body(pos_ref, neg_ref, o_ref):
        lp = jax.nn.log_sigmoid(pos_ref[...])
        ln = jax.nn.log_sigmoid(-neg_ref[...])
        o_ref[0, 0] = -(jnp.sum(lp) + jnp.sum(ln)) / jnp.float32(B)

    out = pl.pallas_call(
        body,
        out_shape=jax.ShapeDtypeStruct((1, 1), jnp.float32),
        out_specs=pl.BlockSpec(memory_space=pltpu.SMEM),
    )(pos2d, neg2d)
    return out[0, 0]


def kernel(center, target, negative, v_emb, u_emb):
    B = center.shape[0]
    V, D = v_emb.shape
    K = negative.shape[1]

    info = plsc.get_sparse_core_info()
    nc, ns = info.num_cores, info.num_subcores
    nw = nc * ns

    center = center.astype(jnp.int32)
    target = target.astype(jnp.int32)
    negr = negative.astype(jnp.int32).reshape(B * K)

    H = V
    v_pack = jnp.pad(v_emb, ((0, 0), (0, D)))
    u_pack = jnp.pad(u_emb, ((0, 0), (0, D)))

    pos, negdots = _sc_dots(center, target, negr, v_pack, u_pack,
                            B, K, D, H, nw, nc)

    pos2d = pos.reshape(B // 128, 128)
    neg2d = negdots.reshape(B * K // 128, 128)
    return _tc_loss(pos2d, neg2d, B)
